# baseline (speedup 1.0000x reference)
import dataclasses
import functools

import jax
import jax.numpy as jnp
from jax import lax
from jax.experimental import pallas as pl
from jax.experimental.pallas import tpu as pltpu
from jax.experimental.pallas import tpu_sc as plsc

UNROLL = 16
NBUF = 4


def _gather_rows(B, C, N, M):
  info = plsc.get_sparse_core_info()
  NC, NS = info.num_cores, info.num_subcores
  NW = NC * NS
  ROWS = B * C
  assert ROWS % NW == 0
  RPW = ROWS // NW
  assert (C % RPW == 0) or (RPW % C == 0)
  assert RPW % NBUF == 0 and M % (16 * UNROLL) == 0

  mesh = plsc.VectorSubcoreMesh(core_axis_name="c", subcore_axis_name="s")

  cp = pltpu.CompilerParams()
  if "needs_layout_passes" in pltpu.CompilerParams.__dataclass_fields__:
    cp = dataclasses.replace(cp, needs_layout_passes=False)

  @functools.partial(
      pl.kernel,
      compiler_params=cp,
      out_type=jax.ShapeDtypeStruct((ROWS, M), jnp.float32),
      mesh=mesh,
      scratch_types=(
          [pltpu.VMEM((M,), jnp.int32)]
          + [pltpu.VMEM((N,), jnp.float32)] * NBUF
          + [pltpu.VMEM((M,), jnp.float32)] * NBUF
          + [pltpu.SemaphoreType.DMA((NBUF,)),
             pltpu.SemaphoreType.DMA((NBUF,))]
      ),
  )
  def k(f_hbm, i_hbm, o_hbm, idx_v, *bufs):
    rows = list(bufs[:NBUF])
    outs = list(bufs[NBUF:2 * NBUF])
    sem_r, sem_o = bufs[2 * NBUF], bufs[2 * NBUF + 1]

    wid = lax.axis_index("s") * NC + lax.axis_index("c")
    r0 = wid * RPW
    b = r0 // C

    pltpu.sync_copy(i_hbm.at[b], idx_v)

    for p in range(NBUF):
      pltpu.async_copy(f_hbm.at[r0 + p], rows[p], sem_r.at[p])

    @pl.loop(0, RPW, step=NBUF)
    def _(g):
      for p in range(NBUF):
        r = g + p
        pltpu.make_async_copy(f_hbm.at[r0 + r], rows[p], sem_r.at[p]).wait()

        @pl.when(r >= NBUF)
        def _():
          pltpu.make_async_copy(outs[p], o_hbm.at[r0 + r - NBUF],
                                sem_o.at[p]).wait()

        @plsc.parallel_loop(0, M, step=16, unroll=UNROLL)
        def _(j):
          s = pl.ds(j, 16)
          outs[p][s] = plsc.load_gather(rows[p], [idx_v[s]])

        pltpu.async_copy(outs[p], o_hbm.at[r0 + r], sem_o.at[p])

        @pl.when(r + NBUF < RPW)
        def _():
          pltpu.async_copy(f_hbm.at[r0 + r + NBUF], rows[p], sem_r.at[p])

    for p in range(NBUF):
      pltpu.make_async_copy(outs[p], o_hbm.at[r0 + RPW - NBUF + p],
                            sem_o.at[p]).wait()

  return k


@jax.jit
def kernel(features, indices):
  B, C, N = features.shape
  M = indices.shape[1]
  k = _gather_rows(B, C, N, M)
  out = k(features.reshape(B * C, N), indices)
  return out.reshape(B, C, M)
